# baseline (speedup 1.0000x reference)
import functools

import jax
import jax.numpy as jnp
from jax.experimental import pallas as pl
from jax.experimental.pallas import tpu as pltpu


_MAX_ITERS = 16
_SLICES = 64


def _entmax_block(z_ref, out_ref):
    z = z_ref[...]
    rows, n = z.shape
    chunks = n // 128
    slice_w = n // _SLICES
    parts = []
    for j in range(_SLICES):
        acc = z[:, j * slice_w:j * slice_w + 128]
        for c in range(1, slice_w // 128):
            acc = jnp.maximum(
                acc, z[:, j * slice_w + c * 128:j * slice_w + (c + 1) * 128])
        parts.append(acc)
    msub = jnp.concatenate(parts, axis=1)
    m1 = functools.reduce(jnp.maximum, parts)
    zmax = jnp.max(m1, axis=1, keepdims=True)
    t0 = zmax - 1.0

    def make_step(data):
        def lstep(_, tt):
            mm = data > tt
            cc = jnp.sum(jnp.where(mm, 1.0, 0.0), axis=1, keepdims=True)
            ss = jnp.sum(jnp.where(mm, data, 0.0), axis=1, keepdims=True)
            return (ss - 1.0) / jnp.maximum(cc, 1.0)
        return lstep

    t_tiny = jax.lax.fori_loop(0, 8, make_step(m1), t0)
    t_warm = jax.lax.fori_loop(0, 3, make_step(msub), t_tiny)

    zeros = jnp.zeros_like(t_warm)

    def cond(carry):
        i, t, t_prev, _, _, _ = carry
        return jnp.logical_and(i < _MAX_ITERS, jnp.any(t != t_prev))

    def step(carry):
        i, t, _, _, _, _ = carry
        m = z > t
        c = jnp.sum(jnp.where(m, 1.0, 0.0), axis=1, keepdims=True)
        s = jnp.sum(jnp.where(m, z, 0.0), axis=1, keepdims=True)
        zx = jnp.max(jnp.where(m, -jnp.inf, z), axis=1, keepdims=True)
        return i + 1, (s - 1.0) / jnp.maximum(c, 1.0), t, c, s, zx

    _, t, _, cf, s, z_next = jax.lax.while_loop(
        cond, step, (0, t_warm, t_warm - 1.0, zeros, zeros, zeros))

    ci = cf.astype(jnp.int32)
    gathered = s + jnp.where(ci < n, z_next, 0.0)
    tau = (gathered - 1.0) / cf
    col = jax.lax.broadcasted_iota(jnp.int32, (rows, n), 1)
    tau_full = jnp.where(col < ci, tau, 0.0)
    r = jnp.maximum(z - tau_full, 0.0)
    out_ref[...] = (r * r) * jax.lax.rsqrt(jnp.maximum(r, 1e-30))


@jax.jit
def kernel(z):
    b, n = z.shape
    rows = 64
    return pl.pallas_call(
        _entmax_block,
        out_shape=jax.ShapeDtypeStruct((b, n), z.dtype),
        grid=(b // rows,),
        in_specs=[pl.BlockSpec((rows, n), lambda i: (i, 0))],
        out_specs=pl.BlockSpec((rows, n), lambda i: (i, 0)),
        compiler_params=pltpu.CompilerParams(
            dimension_semantics=("parallel",),
        ),
    )(z)


# speedup vs baseline: 1.0011x; 1.0011x over previous
import functools

import jax
import jax.numpy as jnp
from jax.experimental import pallas as pl
from jax.experimental.pallas import tpu as pltpu


_MAX_ITERS = 16
_SLICES = 64


def _entmax_block(z_ref, out_ref):
    z = z_ref[...]
    rows, n = z.shape
    slice_w = n // _SLICES
    parts = []
    for j in range(_SLICES):
        acc = z[:, j * slice_w:j * slice_w + 128]
        for c in range(1, slice_w // 128):
            acc = jnp.maximum(
                acc, z[:, j * slice_w + c * 128:j * slice_w + (c + 1) * 128])
        parts.append(acc)
    msub = jnp.concatenate(parts, axis=1)
    m1 = functools.reduce(jnp.maximum, parts)
    zmax = jnp.max(m1, axis=1, keepdims=True)
    t0 = zmax - 1.0

    def make_step(data):
        def lstep(_, tt):
            mm = data > tt
            cc = jnp.sum(jnp.where(mm, 1.0, 0.0), axis=1, keepdims=True)
            ss = jnp.sum(jnp.where(mm, data, 0.0), axis=1, keepdims=True)
            return (ss - 1.0) / jnp.maximum(cc, 1.0)
        return lstep

    t_tiny = jax.lax.fori_loop(0, 8, make_step(m1), t0)
    t_warm = jax.lax.fori_loop(0, 3, make_step(msub), t_tiny)

    zeros = jnp.zeros_like(t_warm)

    def cond(carry):
        i, t, t_prev, _, _, _ = carry
        return jnp.logical_and(i < _MAX_ITERS, jnp.any(t != t_prev))

    def step(carry):
        i, t, _, _, _, _ = carry
        m = z > t
        c = jnp.sum(jnp.where(m, 1.0, 0.0), axis=1, keepdims=True)
        s = jnp.sum(jnp.where(m, z, 0.0), axis=1, keepdims=True)
        zx = jnp.max(jnp.where(m, -jnp.inf, z), axis=1, keepdims=True)
        return i + 1, (s - 1.0) / jnp.maximum(c, 1.0), t, c, s, zx

    _, t, _, cf, s, z_next = jax.lax.while_loop(
        cond, step, (0, t_warm, t_warm - 1.0, zeros, zeros, zeros))

    ci = cf.astype(jnp.int32)
    gathered = s + jnp.where(ci < n, z_next, 0.0)
    tau = (gathered - 1.0) / cf
    col = jax.lax.broadcasted_iota(jnp.int32, (rows, n), 1)
    tau_full = jnp.where(col < ci, tau, 0.0)
    r = jnp.maximum(z - tau_full, 0.0)
    out_ref[...] = (r * r) * jax.lax.rsqrt(jnp.maximum(r, 1e-30))


@jax.jit
def kernel(z):
    b, n = z.shape
    rows = 64
    return pl.pallas_call(
        _entmax_block,
        out_shape=jax.ShapeDtypeStruct((b, n), z.dtype),
        grid=(b // rows,),
        in_specs=[pl.BlockSpec((rows, n), lambda i: (i, 0))],
        out_specs=pl.BlockSpec((rows, n), lambda i: (i, 0)),
        compiler_params=pltpu.CompilerParams(
            dimension_semantics=("parallel",),
        ),
    )(z)
